# Initial kernel scaffold; baseline (speedup 1.0000x reference)
import jax
import jax.numpy as jnp
from jax.experimental import pallas as pl


def kernel(node_weight, edge_index, edge_weight, W1, al1, ar1, b1, g1, be1, W2, al2, ar2, b2, g2, be2, W3, al3, ar3, b3, g3, be3):
    raise NotImplementedError("write your pallas kernel here")


# speedup vs baseline: 1.7573x; 1.7573x over previous
import functools

import jax
import jax.numpy as jnp
from jax.experimental import pallas as pl
from jax.experimental.pallas import tpu as pltpu

N = 10000
D = 128
ROWS = 400


def _dense_body(h_ref, w_ref, al_ref, ar_ref, feat_ref, el_ref, er_ref):
    feat = jnp.dot(h_ref[...], w_ref[...], preferred_element_type=jnp.float32)
    feat_ref[...] = feat
    el_ref[...] = jnp.sum(feat * al_ref[...], axis=1, keepdims=True)
    er_ref[...] = jnp.sum(feat * ar_ref[...], axis=1, keepdims=True)


def _dense_layer(h, W, al, ar):
    return pl.pallas_call(
        _dense_body,
        grid=(N // ROWS,),
        in_specs=[
            pl.BlockSpec((ROWS, D), lambda i: (i, 0)),
            pl.BlockSpec((D, D), lambda i: (0, 0)),
            pl.BlockSpec((1, D), lambda i: (0, 0)),
            pl.BlockSpec((1, D), lambda i: (0, 0)),
        ],
        out_specs=[
            pl.BlockSpec((ROWS, D), lambda i: (i, 0)),
            pl.BlockSpec((ROWS, 1), lambda i: (i, 0)),
            pl.BlockSpec((ROWS, 1), lambda i: (i, 0)),
        ],
        out_shape=[
            jax.ShapeDtypeStruct((N, D), jnp.float32),
            jax.ShapeDtypeStruct((N, 1), jnp.float32),
            jax.ShapeDtypeStruct((N, 1), jnp.float32),
        ],
    )(h, W, al[None, :], ar[None, :])


def _post_body(num_ref, den_ref, b_ref, g_ref, be_ref, out_ref, *, elu):
    num = num_ref[...]
    den = den_ref[...]
    out = jnp.where(den > 0.0, num / den, 0.0) + b_ref[...]
    mu = jnp.mean(out, axis=0, keepdims=True)
    var = jnp.mean((out - mu) * (out - mu), axis=0, keepdims=True)
    out = (out - mu) * jax.lax.rsqrt(var + 1e-5) * g_ref[...] + be_ref[...]
    if elu:
        out = jnp.where(out > 0.0, out, jnp.exp(jnp.minimum(out, 0.0)) - 1.0)
    out_ref[...] = out


def _post_layer(num, den, b, g, be, elu):
    return pl.pallas_call(
        functools.partial(_post_body, elu=elu),
        out_shape=jax.ShapeDtypeStruct((N, D), jnp.float32),
    )(num, den, b[None, :], g[None, :], be[None, :])


def _edge_phase_xla(feat, el, er, src, dst):
    e = jax.nn.leaky_relu(el[src] + er[dst], negative_slope=0.2)
    ex = jnp.exp(e)
    den = jax.ops.segment_sum(ex, dst, num_segments=N)
    num = jax.ops.segment_sum(feat[src] * ex[:, None], dst, num_segments=N)
    return num, den


def kernel(node_weight, edge_index, edge_weight, W1, al1, ar1, b1, g1, be1,
           W2, al2, ar2, b2, g2, be2, W3, al3, ar3, b3, g3, be3):
    src = edge_index[0]
    dst = edge_index[1]
    h = node_weight
    layers = [(W1, al1, ar1, b1, g1, be1, True),
              (W2, al2, ar2, b2, g2, be2, True),
              (W3, al3, ar3, b3, g3, be3, False)]
    for (W, al, ar, b, g, be, elu) in layers:
        feat, el, er = _dense_layer(h, W, al, ar)
        num, den = _edge_phase_xla(feat, el[:, 0], er[:, 0], src, dst)
        h = _post_layer(num, den[:, None], b, g, be, elu)
    return h


# speedup vs baseline: 12.9383x; 7.3625x over previous
import functools

import jax
import jax.numpy as jnp
from jax import lax
from jax.experimental import pallas as pl
from jax.experimental.pallas import tpu as pltpu
from jax.experimental.pallas import tpu_sc as plsc

N = 10000
D = 128
E = 320000

NC = 2
NS = 16
NW = NC * NS

BLK = 1024
IDX_ROWS = BLK // 128
HALF = 256
EPT = 10240
E_PAD = NW * EPT
ZROWS = 640
N_ACC = NS * ZROWS

ROWS = 400


def _dense_body(h_ref, w_ref, al_ref, ar_ref, feat_ref, el_ref, er_ref):
    feat = jnp.dot(h_ref[...], w_ref[...], preferred_element_type=jnp.float32)
    feat_ref[...] = feat
    el_ref[...] = jnp.sum(feat * al_ref[...], axis=1, keepdims=True)
    er_ref[...] = jnp.sum(feat * ar_ref[...], axis=1, keepdims=True)


def _dense_layer(h, W, al, ar):
    return pl.pallas_call(
        _dense_body,
        grid=(N // ROWS,),
        in_specs=[
            pl.BlockSpec((ROWS, D), lambda i: (i, 0)),
            pl.BlockSpec((D, D), lambda i: (0, 0)),
            pl.BlockSpec((1, D), lambda i: (0, 0)),
            pl.BlockSpec((1, D), lambda i: (0, 0)),
        ],
        out_specs=[
            pl.BlockSpec((ROWS, D), lambda i: (i, 0)),
            pl.BlockSpec((ROWS, 1), lambda i: (i, 0)),
            pl.BlockSpec((ROWS, 1), lambda i: (i, 0)),
        ],
        out_shape=[
            jax.ShapeDtypeStruct((N, D), jnp.float32),
            jax.ShapeDtypeStruct((N, 1), jnp.float32),
            jax.ShapeDtypeStruct((N, 1), jnp.float32),
        ],
    )(h, W, al[None, :], ar[None, :])


def _post_body(num_ref, den_ref, b_ref, g_ref, be_ref, out_ref, *, elu):
    num = num_ref[0] + num_ref[1]
    den = den_ref[0] + den_ref[1]
    out = jnp.where(den > 0.0, num / den, 0.0) + b_ref[...]
    mu = jnp.mean(out, axis=0, keepdims=True)
    var = jnp.mean((out - mu) * (out - mu), axis=0, keepdims=True)
    out = (out - mu) * jax.lax.rsqrt(var + 1e-5) * g_ref[...] + be_ref[...]
    if elu:
        out = jnp.where(out > 0.0, out, jnp.exp(jnp.minimum(out, 0.0)) - 1.0)
    out_ref[...] = out


def _post_layer(num, den, b, g, be, elu):
    return pl.pallas_call(
        functools.partial(_post_body, elu=elu),
        out_shape=jax.ShapeDtypeStruct((N, D), jnp.float32),
    )(num, den, b[None, :], g[None, :], be[None, :])


def _edge_body(feat_h, el_h, er_h, src_h, dst_h, zrow_h, zden_h,
               num_out, den_out,
               num_acc, den_acc,
               src_c, dst_c, els_v, erd_v, ex_v, rows_v,
               sem_rows, sem_s):
    cid = lax.axis_index("c")
    sid = lax.axis_index("s")
    wid = sid * NC + cid

    pltpu.sync_copy(zrow_h, num_acc.at[pl.ds(sid * ZROWS, ZROWS)])
    pltpu.sync_copy(zden_h, den_acc.at[pl.ds(sid * ZROWS, ZROWS)])
    plsc.subcore_barrier()

    rbase_t = (wid * EPT) // 128

    def blk_body(k, carry):
        rbase = pl.multiple_of(rbase_t + k * IDX_ROWS, 8)
        pltpu.sync_copy(src_h.at[pl.ds(rbase, IDX_ROWS)], src_c)
        pltpu.sync_copy(dst_h.at[pl.ds(rbase, IDX_ROWS)], dst_c)
        descs_s = []
        for j in range(IDX_ROWS):
            descs_s.append(pltpu.async_copy(
                el_h.at[src_c.at[j]], els_v.at[pl.ds(j * 128, 128)], sem_s))
            descs_s.append(pltpu.async_copy(
                er_h.at[dst_c.at[j]], erd_v.at[pl.ds(j * 128, 128)], sem_s))
        for d in descs_s:
            d.wait()
        for v in range(BLK // 16):
            sl = pl.ds(v * 16, 16)
            e = els_v[sl] + erd_v[sl]
            e = jnp.where(e >= 0.0, e, 0.2 * e)
            ex_v[sl] = jnp.exp(e)
        for j in range(IDX_ROWS):
            pltpu.sync_copy(ex_v.at[pl.ds(j * 128, 128)],
                            den_acc.at[dst_c.at[j]], add=True)

        for half in range(BLK // HALF):
            jo = half * (HALF // 128)
            descs_r = []
            for j in range(HALF // 128):
                descs_r.append(pltpu.async_copy(
                    feat_h.at[src_c.at[jo + j]],
                    rows_v.at[pl.ds(j * 128, 128)], sem_rows))
            for d in descs_r:
                d.wait()

            def row_body(i, c2):
                s = plsc.load_gather(
                    ex_v, [jnp.zeros((16,), jnp.int32) + (half * HALF + i)])
                for j in range(D // 16):
                    sl = pl.ds(j * 16, 16)
                    rows_v[i, sl] = rows_v[i, sl] * s
                return c2

            lax.fori_loop(0, HALF, row_body, 0)
            for j in range(HALF // 128):
                pltpu.sync_copy(rows_v.at[pl.ds(j * 128, 128)],
                                num_acc.at[dst_c.at[jo + j]], add=True)
        return carry

    lax.fori_loop(0, EPT // BLK, blk_body, 0)
    plsc.subcore_barrier()

    pltpu.sync_copy(num_acc.at[pl.ds(sid * 624, 624)],
                    num_out.at[cid, pl.ds(sid * 624, 624)])

    @pl.when(sid == NS - 1)
    def _():
        pltpu.sync_copy(num_acc.at[pl.ds(624 * NS, N - 624 * NS)],
                        num_out.at[cid, pl.ds(624 * NS, N - 624 * NS)])

    @pl.when(sid < 10)
    def _():
        pltpu.sync_copy(den_acc.at[pl.ds(sid * 1024, 1024)],
                        den_out.at[pl.ds(cid * N_ACC + sid * 1024, 1024)])


@functools.partial(jax.jit, static_argnames=())
def _edge_phase_sc(feat, el, er_p, src2, dst2, zrow, zden):
    mesh = plsc.VectorSubcoreMesh(core_axis_name="c", subcore_axis_name="s")
    f = pl.kernel(
        _edge_body,
        out_type=[
            jax.ShapeDtypeStruct((NC, N, D), jnp.float32),
            jax.ShapeDtypeStruct((NC * N_ACC,), jnp.float32),
        ],
        mesh=mesh,
        compiler_params=pltpu.CompilerParams(needs_layout_passes=False),
        scratch_types=[
            pltpu.VMEM_SHARED((N_ACC, D), jnp.float32),
            pltpu.VMEM_SHARED((N_ACC,), jnp.float32),
            pltpu.VMEM((IDX_ROWS, 128), jnp.int32),
            pltpu.VMEM((IDX_ROWS, 128), jnp.int32),
            pltpu.VMEM((BLK,), jnp.float32),
            pltpu.VMEM((BLK,), jnp.float32),
            pltpu.VMEM((BLK,), jnp.float32),
            pltpu.VMEM((HALF, D), jnp.float32),
            pltpu.SemaphoreType.DMA,
            pltpu.SemaphoreType.DMA,
        ],
    )
    return f(feat, el, er_p, src2, dst2, zrow, zden)


def kernel(node_weight, edge_index, edge_weight, W1, al1, ar1, b1, g1, be1,
           W2, al2, ar2, b2, g2, be2, W3, al3, ar3, b3, g3, be3):
    src = edge_index[0].astype(jnp.int32)
    dst = edge_index[1].astype(jnp.int32)
    pad = E_PAD - E
    src2 = jnp.concatenate([src, jnp.zeros((pad,), jnp.int32)]
                           ).reshape(E_PAD // 128, 128)
    dst2 = jnp.concatenate([dst, jnp.full((pad,), N, jnp.int32)]
                           ).reshape(E_PAD // 128, 128)
    zrow = jnp.zeros((ZROWS, D), jnp.float32)
    zden = jnp.zeros((ZROWS,), jnp.float32)

    h = node_weight
    layers = [(W1, al1, ar1, b1, g1, be1, True),
              (W2, al2, ar2, b2, g2, be2, True),
              (W3, al3, ar3, b3, g3, be3, False)]
    for (W, al, ar, b, g, be, elu) in layers:
        feat, el, er = _dense_layer(h, W, al, ar)
        er_p = jnp.concatenate([er[:, 0], jnp.zeros((16,), jnp.float32)])
        num, den = _edge_phase_sc(feat, el[:, 0], er_p, src2, dst2,
                                  zrow, zden)
        den = den.reshape(NC, N_ACC)[:, :N, None]
        h = _post_layer(num, den, b, g, be, elu)
    return h
